# Initial kernel scaffold; baseline (speedup 1.0000x reference)
import jax
import jax.numpy as jnp
from jax.experimental import pallas as pl


def kernel(point_cls_preds, point_cls_labels):
    raise NotImplementedError("write your pallas kernel here")


# speedup vs baseline: 2.4483x; 2.4483x over previous
import jax
import jax.numpy as jnp
from jax.experimental import pallas as pl

_NUM_CLASS = 3
_ALPHA = 0.25
_GAMMA = 2.0
_LANES = 512
_BLOCK_ROWS = 768


def _focal_sum_kernel(x_ref, lab_ref, loss_ref, pos_ref):
    g = pl.program_id(0)
    x = x_ref[...]
    lab = lab_ref[...]
    rows = x.shape[0]
    row = jax.lax.broadcasted_iota(jnp.int32, x.shape, 0) + g * rows
    col = jax.lax.broadcasted_iota(jnp.int32, x.shape, 1)
    cls = (row * _LANES + col) % _NUM_CLASS + 1
    t = (lab == cls).astype(jnp.float32)
    s = jax.nn.sigmoid(x)
    alpha_w = t * _ALPHA + (1.0 - t) * (1.0 - _ALPHA)
    pt = t * (1.0 - s) + (1.0 - t) * s
    bce = jnp.maximum(x, 0.0) - x * t + jnp.log1p(jnp.exp(-jnp.abs(x)))
    loss_sum = jnp.sum(alpha_w * pt * pt * bce, keepdims=True)
    pos_sum = jnp.sum((lab > 0).astype(jnp.float32), keepdims=True)

    @pl.when(g == 0)
    def _init():
        loss_ref[...] = jnp.zeros_like(loss_ref)
        pos_ref[...] = jnp.zeros_like(pos_ref)

    loss_ref[...] += loss_sum
    pos_ref[...] += pos_sum


def kernel(point_cls_preds, point_cls_labels):
    n = point_cls_labels.shape[0]
    flat = n * _NUM_CLASS
    rows = flat // _LANES
    x = point_cls_preds.reshape(rows, _LANES)
    lab3 = jnp.repeat(point_cls_labels, _NUM_CLASS).reshape(rows, _LANES)
    grid = rows // _BLOCK_ROWS
    out = pl.pallas_call(
        _focal_sum_kernel,
        grid=(grid,),
        in_specs=[
            pl.BlockSpec((_BLOCK_ROWS, _LANES), lambda g: (g, 0)),
            pl.BlockSpec((_BLOCK_ROWS, _LANES), lambda g: (g, 0)),
        ],
        out_specs=[
            pl.BlockSpec((1, 1), lambda g: (0, 0)),
            pl.BlockSpec((1, 1), lambda g: (0, 0)),
        ],
        out_shape=[
            jax.ShapeDtypeStruct((1, 1), jnp.float32),
            jax.ShapeDtypeStruct((1, 1), jnp.float32),
        ],
    )(x, lab3)
    loss_sum, pos3 = out
    pos = pos3[0, 0] / jnp.float32(_NUM_CLASS)
    return loss_sum[0, 0] / jnp.maximum(pos, 1.0)


# speedup vs baseline: 65.0382x; 26.5649x over previous
import jax
import jax.numpy as jnp
from jax.experimental import pallas as pl

_NUM_CLASS = 3
_ALPHA = 0.25
_GAMMA = 2.0
_LANES = 512
_BLOCK_ROWS = 256


def _focal_sum_kernel(xt_ref, lab_ref, loss_ref, pos_ref):
    g = pl.program_id(0)
    lab = lab_ref[...]
    total = None
    for c in range(_NUM_CLASS):
        x = xt_ref[c]
        t = (lab == c + 1).astype(jnp.float32)
        s = jax.nn.sigmoid(x)
        alpha_w = t * _ALPHA + (1.0 - t) * (1.0 - _ALPHA)
        pt = t * (1.0 - s) + (1.0 - t) * s
        bce = jnp.maximum(x, 0.0) - x * t + jnp.log1p(jnp.exp(-jnp.abs(x)))
        l = alpha_w * pt * pt * bce
        total = l if total is None else total + l
    loss_sum = jnp.sum(total, keepdims=True)
    pos_sum = jnp.sum((lab > 0).astype(jnp.float32), keepdims=True)

    @pl.when(g == 0)
    def _init():
        loss_ref[...] = jnp.zeros_like(loss_ref)
        pos_ref[...] = jnp.zeros_like(pos_ref)

    loss_ref[...] += loss_sum
    pos_ref[...] += pos_sum


def kernel(point_cls_preds, point_cls_labels):
    n = point_cls_labels.shape[0]
    rows = n // _LANES
    xt = point_cls_preds.T.reshape(_NUM_CLASS, rows, _LANES)
    lab = point_cls_labels.reshape(rows, _LANES)
    grid = rows // _BLOCK_ROWS
    out = pl.pallas_call(
        _focal_sum_kernel,
        grid=(grid,),
        in_specs=[
            pl.BlockSpec((_NUM_CLASS, _BLOCK_ROWS, _LANES), lambda g: (0, g, 0)),
            pl.BlockSpec((_BLOCK_ROWS, _LANES), lambda g: (g, 0)),
        ],
        out_specs=[
            pl.BlockSpec((1, 1), lambda g: (0, 0)),
            pl.BlockSpec((1, 1), lambda g: (0, 0)),
        ],
        out_shape=[
            jax.ShapeDtypeStruct((1, 1), jnp.float32),
            jax.ShapeDtypeStruct((1, 1), jnp.float32),
        ],
    )(xt, lab)
    loss_sum, pos = out
    return loss_sum[0, 0] / jnp.maximum(pos[0, 0], 1.0)
